# baseline (speedup 1.0000x reference)
import functools

import jax
import jax.numpy as jnp
from jax import lax
from jax.experimental import pallas as pl
from jax.experimental.pallas import tpu as pltpu
from jax.experimental.pallas import tpu_sc as plsc

N_CLS = 1000
N_CTX = 16
CTX_DIM = 512
SEQ_LEN = 77
N_EPISODE = 64
SUF_LEN = SEQ_LEN - 1 - N_CTX
ROW = SEQ_LEN * CTX_DIM
SUF_ROW = SUF_LEN * CTX_DIM
CTX_ROW = N_CTX * CTX_DIM
TOK_PAD = 128

_CTX_B = 40
_SUF_CHUNKS = 5


def _whole_body(ctx_ref, prefix2d_ref, suffix2d_ref, whole2d_ref, bcast_ref, sems):
    copies = []
    copies.append(pltpu.make_async_copy(
        prefix2d_ref,
        whole2d_ref.at[pl.ds(0, N_CLS), pl.ds(0, CTX_DIM)],
        sems.at[0]))
    chunk = N_CLS // _SUF_CHUNKS
    for i in range(_SUF_CHUNKS):
        copies.append(pltpu.make_async_copy(
            suffix2d_ref.at[pl.ds(i * chunk, chunk), :],
            whole2d_ref.at[pl.ds(i * chunk, chunk),
                           pl.ds((1 + N_CTX) * CTX_DIM, SUF_ROW)],
            sems.at[1 + i]))
    for c in copies:
        c.start()
    bcast_ref[...] = jnp.broadcast_to(ctx_ref[...].reshape(CTX_ROW)[None],
                                      (_CTX_B, CTX_ROW))
    for j in range(N_CLS // _CTX_B):
        copies.append(pltpu.make_async_copy(
            bcast_ref,
            whole2d_ref.at[pl.ds(j * _CTX_B, _CTX_B), pl.ds(CTX_DIM, CTX_ROW)],
            sems.at[1 + _SUF_CHUNKS + j]))
    for c in copies[1 + _SUF_CHUNKS:]:
        c.start()
    for c in copies:
        c.wait()


def _build_whole(ctx, prefix2d, suffix2d):
    n_dma = 1 + _SUF_CHUNKS + N_CLS // _CTX_B
    return pl.pallas_call(
        _whole_body,
        in_specs=[
            pl.BlockSpec(memory_space=pltpu.VMEM),
            pl.BlockSpec(memory_space=pl.ANY),
            pl.BlockSpec(memory_space=pl.ANY),
        ],
        out_specs=pl.BlockSpec(memory_space=pl.ANY),
        out_shape=jax.ShapeDtypeStruct((N_CLS, ROW), jnp.float32),
        scratch_shapes=[
            pltpu.VMEM((_CTX_B, CTX_ROW), jnp.float32),
            pltpu.SemaphoreType.DMA((n_dma,)),
        ],
    )(ctx, prefix2d, suffix2d)


_NC = 2
_NS = 16
_NW = _NC * _NS
_EP_PER_W = N_EPISODE // _NW


def _episode_body(ids_pad_ref, cls_ref, prefix2d_ref, suffix2d_ref, ctx1_ref,
                  tok_pad_ref, ep_ref, ep_tok_ref,
                  idx_v, suf_v, pre_v, ctx_v, tok_idx_v, tok_v, sem):
    wid = lax.axis_index("s") * _NC + lax.axis_index("c")
    base = wid * _EP_PER_W
    pltpu.sync_copy(ids_pad_ref.at[wid, pl.ds(0, _EP_PER_W)], idx_v)
    suf_cp = pltpu.async_copy(suffix2d_ref.at[idx_v], suf_v, sem)
    suf_cp.wait()
    pltpu.sync_copy(
        suf_v, ep_ref.at[pl.ds(base, _EP_PER_W), pl.ds((1 + N_CTX) * CTX_DIM, SUF_ROW)])
    pre_cp = pltpu.async_copy(prefix2d_ref.at[idx_v], pre_v, sem)
    pre_cp.wait()
    pltpu.sync_copy(pre_v, ep_ref.at[pl.ds(base, _EP_PER_W), pl.ds(0, CTX_DIM)])
    pltpu.sync_copy(ctx1_ref, ctx_v)
    for j in range(_EP_PER_W):
        pltpu.sync_copy(ctx_v, ep_ref.at[pl.ds(base + j, 1), pl.ds(CTX_DIM, CTX_ROW)])

    @pl.when(wid == 0)
    def _tok():
        pltpu.sync_copy(cls_ref, tok_idx_v)
        tok_cp = pltpu.async_copy(tok_pad_ref.at[tok_idx_v], tok_v, sem)
        tok_cp.wait()
        pltpu.sync_copy(tok_v, ep_tok_ref)


def _episode_call(ids_pad, class_ids, prefix2d, suffix2d, ctx1, tok_pad):
    mesh = plsc.VectorSubcoreMesh(core_axis_name="c", subcore_axis_name="s")
    f = functools.partial(
        pl.kernel,
        mesh=mesh,
        out_type=[
            jax.ShapeDtypeStruct((N_EPISODE, ROW), jnp.float32),
            jax.ShapeDtypeStruct((N_EPISODE, TOK_PAD), jnp.int32),
        ],
        scratch_types=[
            pltpu.VMEM((_EP_PER_W,), jnp.int32),
            pltpu.VMEM((_EP_PER_W, SUF_ROW), jnp.float32),
            pltpu.VMEM((_EP_PER_W, CTX_DIM), jnp.float32),
            pltpu.VMEM((1, CTX_ROW), jnp.float32),
            pltpu.VMEM((N_EPISODE,), jnp.int32),
            pltpu.VMEM((N_EPISODE, TOK_PAD), jnp.int32),
            pltpu.SemaphoreType.DMA,
        ],
    )(_episode_body)
    return f(ids_pad, class_ids, prefix2d, suffix2d, ctx1, tok_pad)


def kernel(ctx, token_prefix, token_suffix, tokenized_prompts, class_ids):
    class_ids = class_ids.astype(jnp.int32)
    prefix2d = token_prefix.reshape(N_CLS, CTX_DIM)
    suffix2d = token_suffix.reshape(N_CLS, SUF_ROW)
    ctx1 = ctx.reshape(1, CTX_ROW)
    ids_pad = (jnp.zeros((_NW, 8), jnp.int32)
               .at[:, :_EP_PER_W].set(class_ids.reshape(_NW, _EP_PER_W)))
    tok_pad = jnp.pad(tokenized_prompts.astype(jnp.int32),
                      ((0, 0), (0, TOK_PAD - SEQ_LEN)))

    whole2d = _build_whole(ctx, prefix2d, suffix2d)
    whole = whole2d.reshape(N_CLS, SEQ_LEN, CTX_DIM)
    ep2d, ep_tok_pad = _episode_call(
        ids_pad, class_ids, prefix2d, suffix2d, ctx1, tok_pad)

    episode_prompts = ep2d.reshape(N_EPISODE, SEQ_LEN, CTX_DIM)
    episode_tokenized = ep_tok_pad[:, :SEQ_LEN].astype(tokenized_prompts.dtype)
    return (episode_prompts, episode_tokenized, whole, tokenized_prompts)


# speedup vs baseline: 12.1780x; 12.1780x over previous
import functools

import jax
import jax.numpy as jnp
from jax import lax
from jax.experimental import pallas as pl
from jax.experimental.pallas import tpu as pltpu
from jax.experimental.pallas import tpu_sc as plsc

N_CLS = 1000
N_CTX = 16
CTX_DIM = 512
SEQ_LEN = 77
N_EPISODE = 64
SUF_LEN = SEQ_LEN - 1 - N_CTX
TOK_PAD = 128

_CLS_B = 40


def _whole_body(ctx_ref, prefix_ref, suffix_ref, out_ref):
    out_ref[:, 0:1, :] = prefix_ref[...]
    out_ref[:, 1:1 + N_CTX, :] = jnp.broadcast_to(
        ctx_ref[...][None], (_CLS_B, N_CTX, CTX_DIM))
    out_ref[:, 1 + N_CTX:SEQ_LEN, :] = suffix_ref[...]


def _build_whole(ctx, token_prefix, token_suffix):
    return pl.pallas_call(
        _whole_body,
        grid=(N_CLS // _CLS_B,),
        in_specs=[
            pl.BlockSpec((N_CTX, CTX_DIM), lambda i: (0, 0)),
            pl.BlockSpec((_CLS_B, 1, CTX_DIM), lambda i: (i, 0, 0)),
            pl.BlockSpec((_CLS_B, SUF_LEN, CTX_DIM), lambda i: (i, 0, 0)),
        ],
        out_specs=pl.BlockSpec((_CLS_B, SEQ_LEN, CTX_DIM), lambda i: (i, 0, 0)),
        out_shape=jax.ShapeDtypeStruct((N_CLS, SEQ_LEN, CTX_DIM), jnp.float32),
        compiler_params=pltpu.CompilerParams(
            dimension_semantics=("arbitrary",)),
    )(ctx, token_prefix, token_suffix)


_NC = 2
_NS = 16
_NW = _NC * _NS
_EP_PER_W = N_EPISODE // _NW


def _gather_body(ids_pad_ref, cls_ref, prefix_ref, suffix_ref, tok_pad_ref,
                 ep_suf_ref, ep_pre_ref, ep_tok_ref,
                 idx_v, suf_v, pre_v, tok_idx_v, tok_v, sem, sem2):
    wid = lax.axis_index("s") * _NC + lax.axis_index("c")
    base = wid * _EP_PER_W
    pltpu.sync_copy(ids_pad_ref.at[wid, pl.ds(0, 16)], idx_v)
    ids_vec = idx_v[...]
    cps = []
    for j in range(_EP_PER_W):
        cid = ids_vec[j]
        cps.append(pltpu.async_copy(suffix_ref.at[cid], suf_v.at[j], sem))
        cps.append(pltpu.async_copy(prefix_ref.at[cid], pre_v.at[j], sem2))
    for c in cps:
        c.wait()
    pltpu.sync_copy(suf_v, ep_suf_ref.at[pl.ds(base, _EP_PER_W), :, :])
    pltpu.sync_copy(pre_v, ep_pre_ref.at[pl.ds(base, _EP_PER_W), :, :])

    @pl.when(wid == 0)
    def _tok():
        pltpu.sync_copy(cls_ref, tok_idx_v)
        tok_cp = pltpu.async_copy(tok_pad_ref.at[tok_idx_v], tok_v, sem)
        tok_cp.wait()
        pltpu.sync_copy(tok_v, ep_tok_ref)


def _gather_call(ids_pad, class_ids, token_prefix, token_suffix, tok_pad):
    mesh = plsc.VectorSubcoreMesh(core_axis_name="c", subcore_axis_name="s")
    f = functools.partial(
        pl.kernel,
        mesh=mesh,
        out_type=[
            jax.ShapeDtypeStruct((N_EPISODE, SUF_LEN, CTX_DIM), jnp.float32),
            jax.ShapeDtypeStruct((N_EPISODE, 1, CTX_DIM), jnp.float32),
            jax.ShapeDtypeStruct((N_EPISODE, TOK_PAD), jnp.int32),
        ],
        scratch_types=[
            pltpu.VMEM((16,), jnp.int32),
            pltpu.VMEM((_EP_PER_W, SUF_LEN, CTX_DIM), jnp.float32),
            pltpu.VMEM((_EP_PER_W, 1, CTX_DIM), jnp.float32),
            pltpu.VMEM((N_EPISODE,), jnp.int32),
            pltpu.VMEM((N_EPISODE, TOK_PAD), jnp.int32),
            pltpu.SemaphoreType.DMA,
            pltpu.SemaphoreType.DMA,
        ],
    )(_gather_body)
    return f(ids_pad, class_ids, token_prefix, token_suffix, tok_pad)


_EP_B = 8


def _ep_body(ctx_ref, pre_ref, suf_ref, out_ref):
    out_ref[:, 0:1, :] = pre_ref[...]
    out_ref[:, 1:1 + N_CTX, :] = jnp.broadcast_to(
        ctx_ref[...][None], (_EP_B, N_CTX, CTX_DIM))
    out_ref[:, 1 + N_CTX:SEQ_LEN, :] = suf_ref[...]


def _assemble_episodes(ctx, ep_pre, ep_suf):
    return pl.pallas_call(
        _ep_body,
        grid=(N_EPISODE // _EP_B,),
        in_specs=[
            pl.BlockSpec((N_CTX, CTX_DIM), lambda i: (0, 0)),
            pl.BlockSpec((_EP_B, 1, CTX_DIM), lambda i: (i, 0, 0)),
            pl.BlockSpec((_EP_B, SUF_LEN, CTX_DIM), lambda i: (i, 0, 0)),
        ],
        out_specs=pl.BlockSpec((_EP_B, SEQ_LEN, CTX_DIM), lambda i: (i, 0, 0)),
        out_shape=jax.ShapeDtypeStruct((N_EPISODE, SEQ_LEN, CTX_DIM), jnp.float32),
        compiler_params=pltpu.CompilerParams(
            dimension_semantics=("arbitrary",)),
    )(ctx, ep_pre, ep_suf)


def kernel(ctx, token_prefix, token_suffix, tokenized_prompts, class_ids):
    class_ids = class_ids.astype(jnp.int32)
    ids_pad = (jnp.zeros((_NW, 16), jnp.int32)
               .at[:, :_EP_PER_W].set(class_ids.reshape(_NW, _EP_PER_W)))
    tok_pad = jnp.pad(tokenized_prompts.astype(jnp.int32),
                      ((0, 0), (0, TOK_PAD - SEQ_LEN)))

    whole = _build_whole(ctx, token_prefix, token_suffix)
    ep_suf, ep_pre, ep_tok_pad = _gather_call(
        ids_pad, class_ids, token_prefix, token_suffix, tok_pad)
    episode_prompts = _assemble_episodes(ctx, ep_pre, ep_suf)

    episode_tokenized = ep_tok_pad[:, :SEQ_LEN].astype(tokenized_prompts.dtype)
    return (episode_prompts, episode_tokenized, whole, tokenized_prompts)


# speedup vs baseline: 34.5878x; 2.8402x over previous
import functools

import jax
import jax.numpy as jnp
from jax import lax
from jax.experimental import pallas as pl
from jax.experimental.pallas import tpu as pltpu
from jax.experimental.pallas import tpu_sc as plsc

N_CLS = 1000
N_CTX = 16
CTX_DIM = 512
SEQ_LEN = 77
N_EPISODE = 64
SUF_LEN = SEQ_LEN - 1 - N_CTX
TOK_PAD = 128

_CLS_B = 40


def _whole_body(ctx_ref, pre_ref, suf_ref, out_ref):
    out_ref[0:1] = pre_ref[...]
    out_ref[1:1 + N_CTX] = jnp.broadcast_to(
        ctx_ref[...][:, None, :], (N_CTX, _CLS_B, CTX_DIM))
    out_ref[1 + N_CTX:SEQ_LEN] = suf_ref[...]


def _build_whole_t(ctx, prefix_t, suffix_t):
    return pl.pallas_call(
        _whole_body,
        grid=(N_CLS // _CLS_B,),
        in_specs=[
            pl.BlockSpec((N_CTX, CTX_DIM), lambda i: (0, 0)),
            pl.BlockSpec((1, _CLS_B, CTX_DIM), lambda i: (0, i, 0)),
            pl.BlockSpec((SUF_LEN, _CLS_B, CTX_DIM), lambda i: (0, i, 0)),
        ],
        out_specs=pl.BlockSpec((SEQ_LEN, _CLS_B, CTX_DIM), lambda i: (0, i, 0)),
        out_shape=jax.ShapeDtypeStruct((SEQ_LEN, N_CLS, CTX_DIM), jnp.float32),
        compiler_params=pltpu.CompilerParams(
            dimension_semantics=("arbitrary",)),
    )(ctx, prefix_t, suffix_t)


_NC = 2
_NS = 16
_NW = _NC * _NS
_SLABS_PER_W = 2


def _gather_body(cls_ref, ctx_ref, prefix_t_ref, suffix_t_ref, tok_pad_ref,
                 ep_t_ref, ep_tok_ref,
                 idx_v, buf_v, buf2_v, ctx_v, tok_v, sem, sem2):
    wid = lax.axis_index("s") * _NC + lax.axis_index("c")
    pltpu.sync_copy(cls_ref, idx_v)

    @pl.when(wid < 30)
    def _suf():
        k = wid * _SLABS_PER_W
        cp0 = pltpu.async_copy(suffix_t_ref.at[k].at[idx_v], buf_v, sem)
        cp1 = pltpu.async_copy(suffix_t_ref.at[k + 1].at[idx_v], buf2_v, sem2)
        cp0.wait()
        pltpu.sync_copy(buf_v, ep_t_ref.at[1 + N_CTX + k])
        cp1.wait()
        pltpu.sync_copy(buf2_v, ep_t_ref.at[2 + N_CTX + k])

    @pl.when(wid == 30)
    def _pre():
        cp = pltpu.async_copy(prefix_t_ref.at[0].at[idx_v], buf_v, sem)
        cp.wait()
        pltpu.sync_copy(buf_v, ep_t_ref.at[0])

    @pl.when(wid == 31)
    def _tok():
        cp = pltpu.async_copy(tok_pad_ref.at[idx_v], tok_v, sem)
        cp.wait()
        pltpu.sync_copy(tok_v, ep_tok_ref)

    @pl.when(wid < N_CTX)
    def _ctx():
        cps = [pltpu.async_copy(ctx_ref.at[wid], ctx_v.at[j], sem)
               for j in range(N_EPISODE)]
        for c in cps:
            c.wait()
        pltpu.sync_copy(ctx_v, ep_t_ref.at[1 + wid])


def _gather_call(class_ids, ctx, prefix_t, suffix_t, tok_pad):
    mesh = plsc.VectorSubcoreMesh(core_axis_name="c", subcore_axis_name="s")
    f = functools.partial(
        pl.kernel,
        mesh=mesh,
        out_type=[
            jax.ShapeDtypeStruct((SEQ_LEN, N_EPISODE, CTX_DIM), jnp.float32),
            jax.ShapeDtypeStruct((N_EPISODE, TOK_PAD), jnp.int32),
        ],
        scratch_types=[
            pltpu.VMEM((N_EPISODE,), jnp.int32),
            pltpu.VMEM((N_EPISODE, CTX_DIM), jnp.float32),
            pltpu.VMEM((N_EPISODE, CTX_DIM), jnp.float32),
            pltpu.VMEM((N_EPISODE, CTX_DIM), jnp.float32),
            pltpu.VMEM((N_EPISODE, TOK_PAD), jnp.int32),
            pltpu.SemaphoreType.DMA,
            pltpu.SemaphoreType.DMA,
        ],
    )(_gather_body)
    return f(class_ids, ctx, prefix_t, suffix_t, tok_pad)


def kernel(ctx, token_prefix, token_suffix, tokenized_prompts, class_ids):
    class_ids = class_ids.astype(jnp.int32)
    prefix_t = jnp.transpose(token_prefix, (1, 0, 2))
    suffix_t = jnp.transpose(token_suffix, (1, 0, 2))
    tok_pad = jnp.pad(tokenized_prompts.astype(jnp.int32),
                      ((0, 0), (0, TOK_PAD - SEQ_LEN)))

    whole_t = _build_whole_t(ctx, prefix_t, suffix_t)
    ep_t, ep_tok_pad = _gather_call(class_ids, ctx, prefix_t, suffix_t, tok_pad)

    whole = jnp.transpose(whole_t, (1, 0, 2))
    episode_prompts = jnp.transpose(ep_t, (1, 0, 2))
    episode_tokenized = ep_tok_pad[:, :SEQ_LEN].astype(tokenized_prompts.dtype)
    return (episode_prompts, episode_tokenized, whole, tokenized_prompts)


# speedup vs baseline: 36.6056x; 1.0583x over previous
import functools

import jax
import jax.numpy as jnp
from jax import lax
from jax.experimental import pallas as pl
from jax.experimental.pallas import tpu as pltpu
from jax.experimental.pallas import tpu_sc as plsc

N_CLS = 1000
N_CTX = 16
CTX_DIM = 512
SEQ_LEN = 77
N_EPISODE = 64
SUF_LEN = SEQ_LEN - 1 - N_CTX
TOK_PAD = 128

_CLS_B = 40


def _whole_body(ctx_ref, pre_ref, suf_ref, out_ref):
    out_ref[0:1] = pre_ref[...]
    out_ref[1:1 + N_CTX] = jnp.broadcast_to(
        ctx_ref[...][:, None, :], (N_CTX, _CLS_B, CTX_DIM))
    out_ref[1 + N_CTX:SEQ_LEN] = suf_ref[...]


def _build_whole_t(ctx, prefix_t, suffix_t):
    return pl.pallas_call(
        _whole_body,
        grid=(N_CLS // _CLS_B,),
        in_specs=[
            pl.BlockSpec((N_CTX, CTX_DIM), lambda i: (0, 0)),
            pl.BlockSpec((1, _CLS_B, CTX_DIM), lambda i: (0, i, 0)),
            pl.BlockSpec((SUF_LEN, _CLS_B, CTX_DIM), lambda i: (0, i, 0)),
        ],
        out_specs=pl.BlockSpec((SEQ_LEN, _CLS_B, CTX_DIM), lambda i: (0, i, 0)),
        out_shape=jax.ShapeDtypeStruct((SEQ_LEN, N_CLS, CTX_DIM), jnp.float32),
        compiler_params=pltpu.CompilerParams(
            dimension_semantics=("arbitrary",)),
    )(ctx, prefix_t, suffix_t)


_NC = 2
_NS = 16
_NW = _NC * _NS
_SLABS_PER_W = 2


def _gather_body(cls_ref, ctx_ref, prefix_t_ref, suffix_t_ref, tok_pad_ref,
                 ep_t_ref, ep_tok_ref,
                 idx_v, buf_v, buf2_v, ctx_v, ctx_row_v, tok_v, sem, sem2):
    wid = lax.axis_index("s") * _NC + lax.axis_index("c")
    pltpu.sync_copy(cls_ref, idx_v)

    @pl.when(wid < 30)
    def _suf():
        k = wid * _SLABS_PER_W
        cp0 = pltpu.async_copy(suffix_t_ref.at[k].at[idx_v], buf_v, sem)
        cp1 = pltpu.async_copy(suffix_t_ref.at[k + 1].at[idx_v], buf2_v, sem2)
        cp0.wait()
        pltpu.sync_copy(buf_v, ep_t_ref.at[1 + N_CTX + k])
        cp1.wait()
        pltpu.sync_copy(buf2_v, ep_t_ref.at[2 + N_CTX + k])

    @pl.when(wid == 30)
    def _pre():
        cp = pltpu.async_copy(prefix_t_ref.at[0].at[idx_v], buf_v, sem)
        cp.wait()
        pltpu.sync_copy(buf_v, ep_t_ref.at[0])

    @pl.when(wid == 31)
    def _tok():
        cp = pltpu.async_copy(tok_pad_ref.at[idx_v], tok_v, sem)
        cp.wait()
        pltpu.sync_copy(tok_v, ep_tok_ref)

    @pl.when(wid < N_CTX)
    def _ctx():
        pltpu.sync_copy(ctx_ref.at[wid], ctx_row_v)
        regs = [ctx_row_v[pl.ds(16 * i, 16)] for i in range(CTX_DIM // 16)]

        def _fill(j, carry):
            for i in range(CTX_DIM // 16):
                ctx_v[j, pl.ds(16 * i, 16)] = regs[i]
            return carry

        lax.fori_loop(0, N_EPISODE, _fill, 0)
        pltpu.sync_copy(ctx_v, ep_t_ref.at[1 + wid])


def _gather_call(class_ids, ctx, prefix_t, suffix_t, tok_pad):
    mesh = plsc.VectorSubcoreMesh(core_axis_name="c", subcore_axis_name="s")
    f = functools.partial(
        pl.kernel,
        mesh=mesh,
        out_type=[
            jax.ShapeDtypeStruct((SEQ_LEN, N_EPISODE, CTX_DIM), jnp.float32),
            jax.ShapeDtypeStruct((N_EPISODE, TOK_PAD), jnp.int32),
        ],
        scratch_types=[
            pltpu.VMEM((N_EPISODE,), jnp.int32),
            pltpu.VMEM((N_EPISODE, CTX_DIM), jnp.float32),
            pltpu.VMEM((N_EPISODE, CTX_DIM), jnp.float32),
            pltpu.VMEM((N_EPISODE, CTX_DIM), jnp.float32),
            pltpu.VMEM((CTX_DIM,), jnp.float32),
            pltpu.VMEM((N_EPISODE, TOK_PAD), jnp.int32),
            pltpu.SemaphoreType.DMA,
            pltpu.SemaphoreType.DMA,
        ],
    )(_gather_body)
    return f(class_ids, ctx, prefix_t, suffix_t, tok_pad)


def kernel(ctx, token_prefix, token_suffix, tokenized_prompts, class_ids):
    class_ids = class_ids.astype(jnp.int32)
    prefix_t = jnp.transpose(token_prefix, (1, 0, 2))
    suffix_t = jnp.transpose(token_suffix, (1, 0, 2))
    tok_pad = jnp.pad(tokenized_prompts.astype(jnp.int32),
                      ((0, 0), (0, TOK_PAD - SEQ_LEN)))

    whole_t = _build_whole_t(ctx, prefix_t, suffix_t)
    ep_t, ep_tok_pad = _gather_call(class_ids, ctx, prefix_t, suffix_t, tok_pad)

    whole = jnp.transpose(whole_t, (1, 0, 2))
    episode_prompts = jnp.transpose(ep_t, (1, 0, 2))
    episode_tokenized = ep_tok_pad[:, :SEQ_LEN].astype(tokenized_prompts.dtype)
    return (episode_prompts, episode_tokenized, whole, tokenized_prompts)


# speedup vs baseline: 37.5229x; 1.0251x over previous
import functools

import jax
import jax.numpy as jnp
from jax import lax
from jax.experimental import pallas as pl
from jax.experimental.pallas import tpu as pltpu
from jax.experimental.pallas import tpu_sc as plsc

N_CLS = 1000
N_CTX = 16
CTX_DIM = 512
SEQ_LEN = 77
N_EPISODE = 64
SUF_LEN = SEQ_LEN - 1 - N_CTX
TOK_PAD = 128

_CLS_B = 40


def _whole_body(ctx_ref, pre_ref, suf_ref, out_ref):
    out_ref[0:1] = jnp.transpose(pre_ref[...], (1, 0, 2))
    out_ref[1:1 + N_CTX] = jnp.broadcast_to(
        ctx_ref[...][:, None, :], (N_CTX, _CLS_B, CTX_DIM))
    out_ref[1 + N_CTX:SEQ_LEN] = suf_ref[...]


def _build_whole_t(ctx, prefix3d, suffix_t):
    return pl.pallas_call(
        _whole_body,
        grid=(N_CLS // _CLS_B,),
        in_specs=[
            pl.BlockSpec((N_CTX, CTX_DIM), lambda i: (0, 0)),
            pl.BlockSpec((_CLS_B, 1, CTX_DIM), lambda i: (i, 0, 0)),
            pl.BlockSpec((SUF_LEN, _CLS_B, CTX_DIM), lambda i: (0, i, 0)),
        ],
        out_specs=pl.BlockSpec((SEQ_LEN, _CLS_B, CTX_DIM), lambda i: (0, i, 0)),
        out_shape=jax.ShapeDtypeStruct((SEQ_LEN, N_CLS, CTX_DIM), jnp.float32),
        compiler_params=pltpu.CompilerParams(
            dimension_semantics=("arbitrary",)),
    )(ctx, prefix3d, suffix_t)


_NC = 2
_NS = 16
_NW = _NC * _NS
_SLABS_PER_W = 2


def _gather_body(cls_ref, ctx_ref, prefix3d_ref, suffix_t_ref, tok_pad_ref,
                 ep_t_ref, ep_tok_ref,
                 idx_v, buf_v, buf2_v, ctx_v, ctx_row_v, tok_v, sem, sem2):
    wid = lax.axis_index("s") * _NC + lax.axis_index("c")
    pltpu.sync_copy(cls_ref, idx_v)

    @pl.when(wid < 30)
    def _suf():
        k = wid * _SLABS_PER_W
        cp0 = pltpu.async_copy(suffix_t_ref.at[k].at[idx_v], buf_v, sem)
        cp1 = pltpu.async_copy(suffix_t_ref.at[k + 1].at[idx_v], buf2_v, sem2)
        cp0.wait()
        pltpu.sync_copy(buf_v, ep_t_ref.at[1 + N_CTX + k])
        cp1.wait()
        pltpu.sync_copy(buf2_v, ep_t_ref.at[2 + N_CTX + k])

    @pl.when(wid == 30)
    def _pre():
        cps = []
        for g in range(N_EPISODE // 16):
            ids_vec = idx_v[pl.ds(16 * g, 16)]
            for j in range(16):
                e = 16 * g + j
                cps.append(pltpu.async_copy(
                    prefix3d_ref.at[ids_vec[j]], buf_v.at[pl.ds(e, 1)], sem))
        for c in cps:
            c.wait()
        pltpu.sync_copy(buf_v, ep_t_ref.at[0])

    @pl.when(wid == 31)
    def _tok():
        cp = pltpu.async_copy(tok_pad_ref.at[idx_v], tok_v, sem)
        cp.wait()
        pltpu.sync_copy(tok_v, ep_tok_ref)

    @pl.when(wid < N_CTX)
    def _ctx():
        pltpu.sync_copy(ctx_ref.at[wid], ctx_row_v)
        regs = [ctx_row_v[pl.ds(16 * i, 16)] for i in range(CTX_DIM // 16)]

        def _fill(j, carry):
            for i in range(CTX_DIM // 16):
                ctx_v[j, pl.ds(16 * i, 16)] = regs[i]
            return carry

        lax.fori_loop(0, N_EPISODE, _fill, 0)
        pltpu.sync_copy(ctx_v, ep_t_ref.at[1 + wid])


def _gather_call(class_ids, ctx, prefix3d, suffix_t, tok_pad):
    mesh = plsc.VectorSubcoreMesh(core_axis_name="c", subcore_axis_name="s")
    f = functools.partial(
        pl.kernel,
        mesh=mesh,
        out_type=[
            jax.ShapeDtypeStruct((SEQ_LEN, N_EPISODE, CTX_DIM), jnp.float32),
            jax.ShapeDtypeStruct((N_EPISODE, TOK_PAD), jnp.int32),
        ],
        scratch_types=[
            pltpu.VMEM((N_EPISODE,), jnp.int32),
            pltpu.VMEM((N_EPISODE, CTX_DIM), jnp.float32),
            pltpu.VMEM((N_EPISODE, CTX_DIM), jnp.float32),
            pltpu.VMEM((N_EPISODE, CTX_DIM), jnp.float32),
            pltpu.VMEM((CTX_DIM,), jnp.float32),
            pltpu.VMEM((N_EPISODE, TOK_PAD), jnp.int32),
            pltpu.SemaphoreType.DMA,
            pltpu.SemaphoreType.DMA,
        ],
    )(_gather_body)
    return f(class_ids, ctx, prefix3d, suffix_t, tok_pad)


def kernel(ctx, token_prefix, token_suffix, tokenized_prompts, class_ids):
    class_ids = class_ids.astype(jnp.int32)
    suffix_t = jnp.transpose(token_suffix, (1, 0, 2))
    tok_pad = jnp.pad(tokenized_prompts.astype(jnp.int32),
                      ((0, 0), (0, TOK_PAD - SEQ_LEN)))

    whole_t = _build_whole_t(ctx, token_prefix, suffix_t)
    ep_t, ep_tok_pad = _gather_call(class_ids, ctx, token_prefix, suffix_t,
                                    tok_pad)

    whole = jnp.transpose(whole_t, (1, 0, 2))
    episode_prompts = jnp.transpose(ep_t, (1, 0, 2))
    episode_tokenized = ep_tok_pad[:, :SEQ_LEN].astype(tokenized_prompts.dtype)
    return (episode_prompts, episode_tokenized, whole, tokenized_prompts)
